# baseline (speedup 1.0000x reference)
import functools

import jax
import jax.numpy as jnp
from jax import lax
from jax.experimental import pallas as pl
from jax.experimental.pallas import tpu as pltpu
from jax.experimental.pallas import tpu_sc as plsc

BATCH = 1024
SEQ = 200
EMBED = 128
N = BATCH * SEQ
NW = 32
XROWS = BATCH // NW
PER_W = XROWS * SEQ
C0 = 128
C1 = SEQ - C0
KP = 4
HP = KP // 2


def _emb(x_hbm, table_hbm, out_hbm, idx0, idx1, rows_v, gsem, wsem, isem):
    nc = 2
    wid = lax.axis_index("s") * nc + lax.axis_index("c")
    row0 = wid * XROWS
    base = wid * PER_W
    for tr in range(XROWS // 8):
        pltpu.async_copy(
            x_hbm.at[pl.ds(row0 + tr * 8, 8), pl.ds(0, C0)],
            idx0.at[pl.ds(tr * 8, 8)], isem)
        pltpu.async_copy(
            x_hbm.at[pl.ds(row0 + tr * 8, 8), pl.ds(C0, C1)],
            idx1.at[pl.ds(tr * 8, 8)], isem)
    for tr in range(XROWS // 8):
        pltpu.make_async_copy(
            x_hbm.at[pl.ds(row0, 8), pl.ds(0, C0)],
            idx0.at[pl.ds(tr * 8, 8)], isem).wait()
        pltpu.make_async_copy(
            x_hbm.at[pl.ds(row0, 8), pl.ds(C0, C1)],
            idx1.at[pl.ds(tr * 8, 8)], isem).wait()

    def gather(p, b):
        pltpu.async_copy(
            table_hbm.at[idx0.at[p]],
            rows_v.at[pl.ds(b * SEQ, C0)], gsem.at[b])
        pltpu.async_copy(
            table_hbm.at[idx1.at[p]],
            rows_v.at[pl.ds(b * SEQ + C0, C1)], gsem.at[b])

    def wait_gather(b):
        pltpu.make_async_copy(
            table_hbm.at[idx0.at[0]],
            rows_v.at[pl.ds(b * SEQ, C0)], gsem.at[b]).wait()
        pltpu.make_async_copy(
            table_hbm.at[idx1.at[0]],
            rows_v.at[pl.ds(b * SEQ + C0, C1)], gsem.at[b]).wait()

    def write(p, b):
        return pltpu.async_copy(
            rows_v.at[pl.ds(b * SEQ, SEQ)],
            out_hbm.at[pl.ds(base + p * SEQ, SEQ)], wsem.at[b])

    def wait_write(b):
        pltpu.make_async_copy(
            rows_v.at[pl.ds(b * SEQ, SEQ)],
            out_hbm.at[pl.ds(base, SEQ)], wsem.at[b]).wait()

    for b in range(KP):
        gather(b, b)

    def body(p, carry):
        b = lax.rem(p, KP)
        wait_gather(b)
        write(p, b)
        r = p - HP
        rb = lax.rem(p + HP, KP)

        @pl.when((r >= 0) & (r + KP < XROWS))
        def _():
            wait_write(rb)
            gather(r + KP, rb)

        return carry

    lax.fori_loop(0, XROWS, body, 0)

    def drain(b, carry):
        wait_write(b)
        return carry

    lax.fori_loop(0, KP, drain, 0)


@jax.jit
def _lookup(x, table):
    mesh = plsc.VectorSubcoreMesh(core_axis_name="c", subcore_axis_name="s")
    return pl.kernel(
        _emb,
        out_type=jax.ShapeDtypeStruct((N, EMBED), jnp.float32),
        mesh=mesh,
        scratch_types=[
            pltpu.VMEM((XROWS, C0), jnp.int32),
            pltpu.VMEM((XROWS, C1), jnp.int32),
            pltpu.VMEM((KP * SEQ, EMBED), jnp.float32),
            pltpu.SemaphoreType.DMA((KP,)),
            pltpu.SemaphoreType.DMA((KP,)),
            pltpu.SemaphoreType.DMA,
        ],
    )(x, table)


def kernel(x, hidden_h, hidden_c, table):
    embeds = _lookup(x, table).reshape(BATCH, SEQ, EMBED)
    return (embeds, hidden_h, hidden_c)


# speedup vs baseline: 1.0026x; 1.0026x over previous
import functools

import jax
import jax.numpy as jnp
from jax import lax
from jax.experimental import pallas as pl
from jax.experimental.pallas import tpu as pltpu
from jax.experimental.pallas import tpu_sc as plsc

BATCH = 1024
SEQ = 200
EMBED = 128
N = BATCH * SEQ
NW = 32
XROWS = BATCH // NW
PER_W = XROWS * SEQ
C0 = 128
C1 = SEQ - C0
KP = 4
HP = KP // 2


def _emb(x_hbm, table_hbm, out_hbm, idx0, idx1, rows_v, gsem, wsem, isem):
    nc = 2
    wid = lax.axis_index("s") * nc + lax.axis_index("c")
    row0 = wid * XROWS
    base = wid * PER_W
    for tr in range(XROWS // 8):
        pltpu.async_copy(
            x_hbm.at[pl.ds(row0 + tr * 8, 8), pl.ds(0, C0)],
            idx0.at[pl.ds(tr * 8, 8)], isem)
        pltpu.async_copy(
            x_hbm.at[pl.ds(row0 + tr * 8, 8), pl.ds(C0, C1)],
            idx1.at[pl.ds(tr * 8, 8)], isem)
    for tr in range(XROWS // 8):
        pltpu.make_async_copy(
            x_hbm.at[pl.ds(row0, 8), pl.ds(0, C0)],
            idx0.at[pl.ds(tr * 8, 8)], isem).wait()
        pltpu.make_async_copy(
            x_hbm.at[pl.ds(row0, 8), pl.ds(C0, C1)],
            idx1.at[pl.ds(tr * 8, 8)], isem).wait()

    def gather(p, b):
        pltpu.async_copy(
            table_hbm.at[idx0.at[p, pl.ds(0, C0 // 2)]],
            rows_v.at[pl.ds(b * SEQ, C0 // 2)], gsem.at[b])
        pltpu.async_copy(
            table_hbm.at[idx0.at[p, pl.ds(C0 // 2, C0 // 2)]],
            rows_v.at[pl.ds(b * SEQ + C0 // 2, C0 // 2)], gsem.at[b])
        pltpu.async_copy(
            table_hbm.at[idx1.at[p]],
            rows_v.at[pl.ds(b * SEQ + C0, C1)], gsem.at[b])

    def wait_gather(b):
        pltpu.make_async_copy(
            table_hbm.at[idx0.at[0, pl.ds(0, C0 // 2)]],
            rows_v.at[pl.ds(b * SEQ, C0 // 2)], gsem.at[b]).wait()
        pltpu.make_async_copy(
            table_hbm.at[idx0.at[0, pl.ds(0, C0 // 2)]],
            rows_v.at[pl.ds(b * SEQ + C0 // 2, C0 // 2)], gsem.at[b]).wait()
        pltpu.make_async_copy(
            table_hbm.at[idx1.at[0]],
            rows_v.at[pl.ds(b * SEQ + C0, C1)], gsem.at[b]).wait()

    def write(p, b):
        return pltpu.async_copy(
            rows_v.at[pl.ds(b * SEQ, SEQ)],
            out_hbm.at[pl.ds(base + p * SEQ, SEQ)], wsem.at[b])

    def wait_write(b):
        pltpu.make_async_copy(
            rows_v.at[pl.ds(b * SEQ, SEQ)],
            out_hbm.at[pl.ds(base, SEQ)], wsem.at[b]).wait()

    for b in range(KP):
        gather(b, b)

    def body(p, carry):
        b = lax.rem(p, KP)
        wait_gather(b)
        write(p, b)
        r = p - HP
        rb = lax.rem(p + HP, KP)

        @pl.when((r >= 0) & (r + KP < XROWS))
        def _():
            wait_write(rb)
            gather(r + KP, rb)

        return carry

    lax.fori_loop(0, XROWS, body, 0)

    def drain(b, carry):
        wait_write(b)
        return carry

    lax.fori_loop(0, KP, drain, 0)


@jax.jit
def _lookup(x, table):
    mesh = plsc.VectorSubcoreMesh(core_axis_name="c", subcore_axis_name="s")
    return pl.kernel(
        _emb,
        out_type=jax.ShapeDtypeStruct((N, EMBED), jnp.float32),
        mesh=mesh,
        scratch_types=[
            pltpu.VMEM((XROWS, C0), jnp.int32),
            pltpu.VMEM((XROWS, C1), jnp.int32),
            pltpu.VMEM((KP * SEQ, EMBED), jnp.float32),
            pltpu.SemaphoreType.DMA((KP,)),
            pltpu.SemaphoreType.DMA((KP,)),
            pltpu.SemaphoreType.DMA,
        ],
    )(x, table)


def kernel(x, hidden_h, hidden_c, table):
    embeds = _lookup(x, table).reshape(BATCH, SEQ, EMBED)
    return (embeds, hidden_h, hidden_c)


# speedup vs baseline: 1.0116x; 1.0091x over previous
import functools

import jax
import jax.numpy as jnp
from jax import lax
from jax.experimental import pallas as pl
from jax.experimental.pallas import tpu as pltpu
from jax.experimental.pallas import tpu_sc as plsc

BATCH = 1024
SEQ = 200
EMBED = 128
N = BATCH * SEQ
NW = 32
XROWS = BATCH // NW
PER_W = XROWS * SEQ
C0 = 128
C1 = SEQ - C0
SEG = (64, 64, 40, 32)
OFF = (0, 64, 128, 168)
NSEG = 4
NCH = XROWS * NSEG
K = 12
SLOT = 64
H = 6


def _emb(x_hbm, table_hbm, out_hbm, idx0, idx1, rows_v, gsem, wsem, isem):
    nc = 2
    wid = lax.axis_index("s") * nc + lax.axis_index("c")
    row0 = wid * XROWS
    base = wid * PER_W
    for tr in range(XROWS // 8):
        pltpu.async_copy(
            x_hbm.at[pl.ds(row0 + tr * 8, 8), pl.ds(0, C0)],
            idx0.at[pl.ds(tr * 8, 8)], isem)
        pltpu.async_copy(
            x_hbm.at[pl.ds(row0 + tr * 8, 8), pl.ds(C0, C1)],
            idx1.at[pl.ds(tr * 8, 8)], isem)
    for tr in range(XROWS // 8):
        pltpu.make_async_copy(
            x_hbm.at[pl.ds(row0, 8), pl.ds(0, C0)],
            idx0.at[pl.ds(tr * 8, 8)], isem).wait()
        pltpu.make_async_copy(
            x_hbm.at[pl.ds(row0, 8), pl.ds(C0, C1)],
            idx1.at[pl.ds(tr * 8, 8)], isem).wait()

    def idx_src(q, seg):
        if seg == 0:
            return idx0.at[q, pl.ds(0, 64)]
        if seg == 1:
            return idx0.at[q, pl.ds(64, 64)]
        if seg == 2:
            return idx1.at[q, pl.ds(0, 40)]
        return idx1.at[q, pl.ds(40, 32)]

    def gather(q, seg, b):
        pltpu.async_copy(
            table_hbm.at[idx_src(q, seg)],
            rows_v.at[pl.ds(b * SLOT, SEG[seg])], gsem.at[b])

    def wait_gather(seg, b):
        pltpu.make_async_copy(
            table_hbm.at[idx_src(0, seg)],
            rows_v.at[pl.ds(b * SLOT, SEG[seg])], gsem.at[b]).wait()

    def write(q, seg, b):
        pltpu.async_copy(
            rows_v.at[pl.ds(b * SLOT, SEG[seg])],
            out_hbm.at[pl.ds(base + q * SEQ + OFF[seg], SEG[seg])],
            wsem.at[b])

    def wait_write(seg, b):
        pltpu.make_async_copy(
            rows_v.at[pl.ds(b * SLOT, SEG[seg])],
            out_hbm.at[pl.ds(base, SEG[seg])], wsem.at[b]).wait()

    for j in range(K):
        gather(j // NSEG, j % NSEG, j)

    def body(q, carry):
        for i in range(NSEG):
            b = lax.rem(q * NSEG + i, K)
            wait_gather(i, b)
            write(q, i, b)
            rj = q * NSEG + i - H
            rseg = (i + 2) % NSEG
            rq = q + 1 + (i + 2) // NSEG
            rb = lax.rem(rj, K)

            @pl.when((rj >= 0) & (rq < XROWS))
            def _():
                wait_write(rseg, rb)
                gather(rq, rseg, rb)

        return carry

    lax.fori_loop(0, XROWS, body, 0)

    def drain(s, carry):
        for i in range(NSEG):
            wait_write(i, lax.rem(s * NSEG + i, K))
        return carry

    lax.fori_loop(NCH // NSEG - K // NSEG, NCH // NSEG, drain, 0)


@jax.jit
def _lookup(x, table):
    mesh = plsc.VectorSubcoreMesh(core_axis_name="c", subcore_axis_name="s")
    return pl.kernel(
        _emb,
        out_type=jax.ShapeDtypeStruct((N, EMBED), jnp.float32),
        mesh=mesh,
        scratch_types=[
            pltpu.VMEM((XROWS, C0), jnp.int32),
            pltpu.VMEM((XROWS, C1), jnp.int32),
            pltpu.VMEM((K * SLOT, EMBED), jnp.float32),
            pltpu.SemaphoreType.DMA((K,)),
            pltpu.SemaphoreType.DMA((K,)),
            pltpu.SemaphoreType.DMA,
        ],
    )(x, table)


def kernel(x, hidden_h, hidden_c, table):
    embeds = _lookup(x, table).reshape(BATCH, SEQ, EMBED)
    return (embeds, hidden_h, hidden_c)


# speedup vs baseline: 1.0128x; 1.0011x over previous
import jax
import jax.numpy as jnp
from jax import lax
from jax.experimental import pallas as pl
from jax.experimental.pallas import tpu as pltpu
from jax.experimental.pallas import tpu_sc as plsc

BATCH = 1024
SEQ = 200
EMBED = 128
N = BATCH * SEQ
NW = 32
PER_W = N // NW
CHUNK = 64
NCH = PER_W // CHUNK
K = 12
H = K // 2


def _emb(idx_hbm, table_hbm, out_hbm, idx_v, rows_v, gsem, wsem):
    nc = 2
    wid = lax.axis_index("s") * nc + lax.axis_index("c")
    base = wid * PER_W
    pltpu.sync_copy(idx_hbm.at[pl.ds(base, PER_W)], idx_v)

    def gather(j, b):
        return pltpu.async_copy(
            table_hbm.at[idx_v.at[pl.ds(j * CHUNK, CHUNK)]],
            rows_v.at[pl.ds(b * CHUNK, CHUNK)], gsem.at[b])

    def wait_gather(b):
        pltpu.make_async_copy(
            table_hbm.at[idx_v.at[pl.ds(0, CHUNK)]],
            rows_v.at[pl.ds(b * CHUNK, CHUNK)], gsem.at[b]).wait()

    def write(j, b):
        return pltpu.async_copy(
            rows_v.at[pl.ds(b * CHUNK, CHUNK)],
            out_hbm.at[pl.ds(base + j * CHUNK, CHUNK)], wsem.at[b])

    def wait_write(b):
        pltpu.make_async_copy(
            rows_v.at[pl.ds(b * CHUNK, CHUNK)],
            out_hbm.at[pl.ds(base, CHUNK)], wsem.at[b]).wait()

    for b in range(K):
        gather(b, b)

    def body(j, carry):
        b = lax.rem(j, K)
        wait_gather(b)
        write(j, b)
        r = j - H
        rb = lax.rem(j + H, K)

        @pl.when((r >= 0) & (r + K < NCH))
        def _():
            wait_write(rb)
            gather(r + K, rb)

        return carry

    lax.fori_loop(0, NCH, body, 0)

    def drain(b, carry):
        wait_write(b)
        return carry

    lax.fori_loop(0, K, drain, 0)


@jax.jit
def _lookup(idx, table):
    mesh = plsc.VectorSubcoreMesh(core_axis_name="c", subcore_axis_name="s")
    return pl.kernel(
        _emb,
        out_type=jax.ShapeDtypeStruct((N, EMBED), jnp.float32),
        mesh=mesh,
        scratch_types=[
            pltpu.VMEM((PER_W,), jnp.int32),
            pltpu.VMEM((K * CHUNK, EMBED), jnp.float32),
            pltpu.SemaphoreType.DMA((K,)),
            pltpu.SemaphoreType.DMA((K,)),
        ],
    )(idx, table)


def kernel(x, hidden_h, hidden_c, table):
    idx = x.reshape(N)
    embeds = _lookup(idx, table).reshape(BATCH, SEQ, EMBED)
    return (embeds, hidden_h, hidden_c)
